# baseline (speedup 1.0000x reference)
import functools
import math

import jax
import jax.numpy as jnp
from jax import lax
from jax.experimental import pallas as pl
from jax.experimental.pallas import tpu as pltpu
from jax.experimental.pallas import tpu_sc as plsc

N = 10000
E = 320000
IN = 128
HID = 128
OUT = 64
L_LAYERS = 8
ALPHA = 0.1
LAM = 0.5

NC = 2
NS = 16
NW = NC * NS
CH = 128
NCHUNK = 80
EPT = NCHUNK * CH
EPAD = NW * EPT
NPAD = N + 8
NSLAB = 10
SLAB = N // NSLAB
ZR = 200
DEGW = 128

_mesh = plsc.VectorSubcoreMesh(
    core_axis_name="c", subcore_axis_name="s", num_cores=NC, num_subcores=NS)


PACK_SHIFT = 14
PACK_MASK = (1 << PACK_SHIFT) - 1


def _unpack_chunk(packed, j, sidx, didx):
    for k in range(CH // 16):
        p = packed[j, pl.ds(16 * k, 16)]
        if sidx is not None:
            sidx[pl.ds(16 * k, 16)] = p >> PACK_SHIFT
        didx[pl.ds(16 * k, 16)] = p & PACK_MASK


def _deg_body(packed_hbm, zeros_hbm, out_hbm, packed_v, didx2, ones_v, deg_s,
              ssem):
    cid = lax.axis_index("c")
    sid = lax.axis_index("s")
    wid = sid * NC + cid

    one16 = jnp.ones((16,), jnp.float32)

    def fill_ones(i, _):
        for k in range(DEGW // 16):
            ones_v[i, pl.ds(16 * k, 16)] = one16
        return 0

    lax.fori_loop(0, CH, fill_ones, 0, unroll=False)

    @pl.when(sid < NSLAB)
    def _():
        pltpu.sync_copy(zeros_hbm, deg_s.at[pl.ds(sid * SLAB, SLAB)])

    pltpu.sync_copy(packed_hbm.at[wid], packed_v)

    def unpack(j, _):
        _unpack_chunk(packed_v, j, None, didx2.at[j])
        return 0

    lax.fori_loop(0, NCHUNK, unpack, 0, unroll=False)
    plsc.subcore_barrier()

    def chunk(j, _):
        pltpu.async_copy(ones_v, deg_s.at[didx2.at[j]], ssem, add=True)
        return 0

    lax.fori_loop(0, NCHUNK, chunk, 0, unroll=False)

    def drain(j, _):
        pltpu.make_async_copy(ones_v, deg_s.at[didx2.at[j]], ssem).wait()
        return 0

    lax.fori_loop(0, NCHUNK, drain, 0, unroll=False)
    plsc.subcore_barrier()

    @pl.when(sid < NSLAB)
    def _():
        pltpu.sync_copy(deg_s.at[pl.ds(sid * SLAB, SLAB)],
                        out_hbm.at[cid, pl.ds(sid * SLAB, SLAB)])


_deg_call = pl.kernel(
    _deg_body,
    out_type=jax.ShapeDtypeStruct((NC, N, DEGW), jnp.float32),
    mesh=_mesh,
    scratch_types=[
        pltpu.VMEM((NCHUNK, CH), jnp.int32),
        pltpu.VMEM((NCHUNK, CH), jnp.int32),
        pltpu.VMEM((CH, DEGW), jnp.float32),
        pltpu.VMEM_SHARED((NPAD, DEGW), jnp.float32),
        pltpu.SemaphoreType.DMA,
    ],
)


def _spmm_body(hs_hbm, packed_hbm, zeros_hbm, out_hbm,
               packed_v, sidx0, sidx1, didx0, didx1, rows0, rows1,
               agg_s, sem0, sem1):
    cid = lax.axis_index("c")
    sid = lax.axis_index("s")
    wid = sid * NC + cid

    @pl.when(sid < NSLAB)
    def _():
        pltpu.sync_copy(zeros_hbm, agg_s.at[pl.ds(sid * SLAB, SLAB)])

    pltpu.sync_copy(packed_hbm.at[wid], packed_v)
    plsc.subcore_barrier()

    rows = (rows0, rows1)
    sidx = (sidx0, sidx1)
    didx = (didx0, didx1)
    sems = (sem0, sem1)

    _unpack_chunk(packed_v, 0, sidx0, didx0)
    pltpu.async_copy(hs_hbm.at[sidx0], rows0, sem0)

    def chunk2(j2, _):
        for b in range(2):
            j = j2 * 2 + b

            @pl.when(j + 1 < NCHUNK)
            def _():
                _unpack_chunk(packed_v, j + 1, sidx[1 - b], didx[1 - b])
                pltpu.async_copy(hs_hbm.at[sidx[1 - b]],
                                 rows[1 - b], sems[1 - b])

            pltpu.make_async_copy(hs_hbm.at[sidx[b]],
                                  rows[b], sems[b]).wait()
            pltpu.sync_copy(rows[b], agg_s.at[didx[b]], add=True)
        return 0

    lax.fori_loop(0, NCHUNK // 2, chunk2, 0, unroll=False)
    plsc.subcore_barrier()

    @pl.when(sid < NSLAB)
    def _():
        pltpu.sync_copy(agg_s.at[pl.ds(sid * SLAB, SLAB)],
                        out_hbm.at[cid, pl.ds(sid * SLAB, SLAB)])


_spmm_call = pl.kernel(
    _spmm_body,
    out_type=jax.ShapeDtypeStruct((NC, N, HID), jnp.float32),
    mesh=_mesh,
    scratch_types=[
        pltpu.VMEM((NCHUNK, CH), jnp.int32),
        pltpu.VMEM((CH,), jnp.int32),
        pltpu.VMEM((CH,), jnp.int32),
        pltpu.VMEM((CH,), jnp.int32),
        pltpu.VMEM((CH,), jnp.int32),
        pltpu.VMEM((CH, HID), jnp.float32),
        pltpu.VMEM((CH, HID), jnp.float32),
        pltpu.VMEM_SHARED((NPAD, HID), jnp.float32),
        pltpu.SemaphoreType.DMA,
        pltpu.SemaphoreType.DMA,
    ],
)


_BLK = 1000
_GRID = N // _BLK


def _proj_body(x_ref, w_ref, b_ref, o_ref):
    o_ref[...] = jnp.maximum(
        jnp.dot(x_ref[...], w_ref[...], preferred_element_type=jnp.float32)
        + b_ref[...], 0.0)


def _proj(x, wt, b):
    return pl.pallas_call(
        _proj_body,
        grid=(_GRID,),
        in_specs=[
            pl.BlockSpec((_BLK, IN), lambda i: (i, 0)),
            pl.BlockSpec((IN, HID), lambda i: (0, 0)),
            pl.BlockSpec((1, HID), lambda i: (0, 0)),
        ],
        out_specs=pl.BlockSpec((_BLK, HID), lambda i: (i, 0)),
        out_shape=jax.ShapeDtypeStruct((N, HID), jnp.float32),
    )(x, wt, b)


def _norm_body(d0_ref, d1_ref, h_ref, n_ref, hs_ref):
    deg = d0_ref[:, 0:1] + d1_ref[:, 0:1]
    nrm = lax.rsqrt(jnp.maximum(deg, 1.0))
    n_ref[...] = nrm
    hs_ref[...] = h_ref[...] * nrm


def _norm(d0, d1, h):
    return pl.pallas_call(
        _norm_body,
        grid=(_GRID,),
        in_specs=[
            pl.BlockSpec((_BLK, DEGW), lambda i: (i, 0)),
            pl.BlockSpec((_BLK, DEGW), lambda i: (i, 0)),
            pl.BlockSpec((_BLK, HID), lambda i: (i, 0)),
        ],
        out_specs=[
            pl.BlockSpec((_BLK, 1), lambda i: (i, 0)),
            pl.BlockSpec((_BLK, HID), lambda i: (i, 0)),
        ],
        out_shape=[
            jax.ShapeDtypeStruct((N, 1), jnp.float32),
            jax.ShapeDtypeStruct((N, HID), jnp.float32),
        ],
    )(d0, d1, h)


def _combine_body(beta, a0_ref, a1_ref, h0_ref, n_ref, w_ref, h_ref, hs_ref):
    agg = (a0_ref[...] + a1_ref[...]) * n_ref[...]
    r = (1.0 - ALPHA) * agg + ALPHA * h0_ref[...]
    r = (1.0 - beta) * r + beta * jnp.dot(
        r, w_ref[...], preferred_element_type=jnp.float32)
    h = jnp.maximum(r, 0.0)
    h_ref[...] = h
    hs_ref[...] = h * n_ref[...]


def _combine(a0, a1, h0, nrm, w, beta):
    return pl.pallas_call(
        functools.partial(_combine_body, beta),
        grid=(_GRID,),
        in_specs=[
            pl.BlockSpec((_BLK, HID), lambda i: (i, 0)),
            pl.BlockSpec((_BLK, HID), lambda i: (i, 0)),
            pl.BlockSpec((_BLK, HID), lambda i: (i, 0)),
            pl.BlockSpec((_BLK, 1), lambda i: (i, 0)),
            pl.BlockSpec((HID, HID), lambda i: (0, 0)),
        ],
        out_specs=[
            pl.BlockSpec((_BLK, HID), lambda i: (i, 0)),
            pl.BlockSpec((_BLK, HID), lambda i: (i, 0)),
        ],
        out_shape=[
            jax.ShapeDtypeStruct((N, HID), jnp.float32),
            jax.ShapeDtypeStruct((N, HID), jnp.float32),
        ],
    )(a0, a1, h0, nrm, w)


def _out_body(h_ref, w_ref, b_ref, o_ref):
    o = jnp.dot(h_ref[...], w_ref[...],
                preferred_element_type=jnp.float32) + b_ref[...]
    m = jnp.max(o, axis=1, keepdims=True)
    e = jnp.exp(o - m)
    lse = jnp.log(jnp.sum(e, axis=1, keepdims=True)) + m
    o_ref[...] = o - lse


def _outproj(h, wt, b):
    return pl.pallas_call(
        _out_body,
        grid=(_GRID,),
        in_specs=[
            pl.BlockSpec((_BLK, HID), lambda i: (i, 0)),
            pl.BlockSpec((HID, OUT), lambda i: (0, 0)),
            pl.BlockSpec((1, OUT), lambda i: (0, 0)),
        ],
        out_specs=pl.BlockSpec((_BLK, OUT), lambda i: (i, 0)),
        out_shape=jax.ShapeDtypeStruct((N, OUT), jnp.float32),
    )(h, wt, b)


def kernel(features, edge_index, fc0_w, fc0_b, fc1_w, fc1_b, conv_w):
    src = edge_index[0]
    dst = edge_index[1]

    pad = EPAD - E
    src_p = jnp.concatenate([src, jnp.zeros((pad,), jnp.int32)])
    dst_p = jnp.concatenate([dst, jnp.full((pad,), N, jnp.int32)])
    packed = ((src_p << PACK_SHIFT) | dst_p).reshape(NW, NCHUNK, CH)
    zeros = jnp.zeros((SLAB, HID), jnp.float32)

    deg2 = _deg_call(packed, zeros)
    h0 = _proj(features, fc0_w.T, fc0_b.reshape(1, HID))
    nrm, hs = _norm(deg2[0], deg2[1], h0)

    h = h0
    for i in range(L_LAYERS):
        agg2 = _spmm_call(hs, packed, zeros)
        beta = math.log(LAM / (i + 1) + 1.0)
        h, hs = _combine(agg2[0], agg2[1], h0, nrm, conv_w[i], beta)

    return _outproj(h, fc1_w.T, fc1_b.reshape(1, OUT))


# speedup vs baseline: 2.2879x; 2.2879x over previous
import functools
import math

import jax
import jax.numpy as jnp
from jax import lax
from jax.experimental import pallas as pl
from jax.experimental.pallas import tpu as pltpu
from jax.experimental.pallas import tpu_sc as plsc

N = 10000
E = 320000
IN = 128
HID = 128
OUT = 64
L_LAYERS = 8
ALPHA = 0.1
LAM = 0.5

NC = 2
NS = 16
NW = NC * NS
CH = 128
NCHUNK = 80
EPT = NCHUNK * CH
EPAD = NW * EPT
NPAD = N + 8
NSLAB = 10
SLAB = N // NSLAB
ZR = 200
DEGW = 128

_mesh = plsc.VectorSubcoreMesh(
    core_axis_name="c", subcore_axis_name="s", num_cores=NC, num_subcores=NS)


PACK_SHIFT = 14
PACK_MASK = (1 << PACK_SHIFT) - 1


def _unpack_chunk(packed, j, sidx, didx):
    for k in range(CH // 16):
        p = packed[j, pl.ds(16 * k, 16)]
        if sidx is not None:
            sidx[pl.ds(16 * k, 16)] = p >> PACK_SHIFT
        didx[pl.ds(16 * k, 16)] = p & PACK_MASK


def _deg_body(packed_hbm, zeros_hbm, out_hbm, packed_v, didx2, ones_v, deg_s,
              ssem):
    cid = lax.axis_index("c")
    sid = lax.axis_index("s")
    wid = sid * NC + cid

    one16 = jnp.ones((16,), jnp.float32)

    def fill_ones(i, _):
        for k in range(DEGW // 16):
            ones_v[i, pl.ds(16 * k, 16)] = one16
        return 0

    lax.fori_loop(0, CH, fill_ones, 0, unroll=False)

    @pl.when(sid < NSLAB)
    def _():
        pltpu.sync_copy(zeros_hbm, deg_s.at[pl.ds(sid * SLAB, SLAB)])

    pltpu.sync_copy(packed_hbm.at[wid], packed_v)

    def unpack(j, _):
        _unpack_chunk(packed_v, j, None, didx2.at[j])
        return 0

    lax.fori_loop(0, NCHUNK, unpack, 0, unroll=False)
    plsc.subcore_barrier()

    def chunk(j, _):
        pltpu.async_copy(ones_v, deg_s.at[didx2.at[j]], ssem, add=True)
        return 0

    lax.fori_loop(0, NCHUNK, chunk, 0, unroll=False)

    def drain(j, _):
        pltpu.make_async_copy(ones_v, deg_s.at[didx2.at[j]], ssem).wait()
        return 0

    lax.fori_loop(0, NCHUNK, drain, 0, unroll=False)
    plsc.subcore_barrier()

    @pl.when(sid < NSLAB)
    def _():
        pltpu.sync_copy(deg_s.at[pl.ds(sid * SLAB, SLAB)],
                        out_hbm.at[cid, pl.ds(sid * SLAB, SLAB)])


_deg_call = pl.kernel(
    _deg_body,
    out_type=jax.ShapeDtypeStruct((NC, N, DEGW), jnp.float32),
    mesh=_mesh,
    scratch_types=[
        pltpu.VMEM((NCHUNK, CH), jnp.int32),
        pltpu.VMEM((NCHUNK, CH), jnp.int32),
        pltpu.VMEM((CH, DEGW), jnp.float32),
        pltpu.VMEM_SHARED((NPAD, DEGW), jnp.float32),
        pltpu.SemaphoreType.DMA,
    ],
)


SCH = 80
SNCHUNK = 10000 // SCH
SEPT = 10000


def _spmm_body(hs_hbm, src_hbm, dst_hbm, zeros_hbm, out_hbm,
               sidx0, sidx1, didx0, didx1, rows0, rows1,
               agg_s, sem0, sem1):
    cid = lax.axis_index("c")
    sid = lax.axis_index("s")
    wid = sid * NC + cid

    @pl.when(sid < NSLAB)
    def _():
        pltpu.sync_copy(zeros_hbm, agg_s.at[pl.ds(sid * SLAB, SLAB)])
    plsc.subcore_barrier()

    rows = (rows0, rows1)
    sidx = (sidx0, sidx1)
    didx = (didx0, didx1)
    sems = (sem0, sem1)
    base = wid * SEPT

    def stage(j, b):
        pltpu.sync_copy(src_hbm.at[pl.ds(base + j * SCH, SCH)], sidx[b])
        pltpu.sync_copy(dst_hbm.at[pl.ds(base + j * SCH, SCH)], didx[b])
        pltpu.async_copy(hs_hbm.at[sidx[b]], rows[b], sems[b])

    def finish(b):
        pltpu.make_async_copy(hs_hbm.at[sidx[b]], rows[b], sems[b]).wait()
        pltpu.sync_copy(rows[b], agg_s.at[didx[b]], add=True)

    stage(0, 0)

    def chunk2(j2, _):
        for b2 in range(2):
            j = 1 + j2 * 2 + b2
            stage(j, (1 + b2) % 2)
            finish(b2)
        return 0

    lax.fori_loop(0, (SNCHUNK - 1) // 2, chunk2, 0, unroll=False)
    finish((SNCHUNK - 1) % 2)
    plsc.subcore_barrier()

    @pl.when(sid < NSLAB)
    def _():
        pltpu.sync_copy(agg_s.at[pl.ds(sid * SLAB, SLAB)],
                        out_hbm.at[cid, pl.ds(sid * SLAB, SLAB)])


_spmm_call = pl.kernel(
    _spmm_body,
    out_type=jax.ShapeDtypeStruct((NC, N, HID), jnp.float32),
    mesh=_mesh,
    scratch_types=[
        pltpu.VMEM((SCH,), jnp.int32),
        pltpu.VMEM((SCH,), jnp.int32),
        pltpu.VMEM((SCH,), jnp.int32),
        pltpu.VMEM((SCH,), jnp.int32),
        pltpu.VMEM((SCH, HID), jnp.float32),
        pltpu.VMEM((SCH, HID), jnp.float32),
        pltpu.VMEM_SHARED((N, HID), jnp.float32),
        pltpu.SemaphoreType.DMA,
        pltpu.SemaphoreType.DMA,
    ],
)


_BLK = 1000
_GRID = N // _BLK


def _proj_body(x_ref, w_ref, b_ref, o_ref):
    o_ref[...] = jnp.maximum(
        jnp.dot(x_ref[...], w_ref[...], preferred_element_type=jnp.float32)
        + b_ref[...], 0.0)


def _proj(x, wt, b):
    return pl.pallas_call(
        _proj_body,
        grid=(_GRID,),
        in_specs=[
            pl.BlockSpec((_BLK, IN), lambda i: (i, 0)),
            pl.BlockSpec((IN, HID), lambda i: (0, 0)),
            pl.BlockSpec((1, HID), lambda i: (0, 0)),
        ],
        out_specs=pl.BlockSpec((_BLK, HID), lambda i: (i, 0)),
        out_shape=jax.ShapeDtypeStruct((N, HID), jnp.float32),
    )(x, wt, b)


def _norm_body(d0_ref, d1_ref, h_ref, n_ref, hs_ref):
    deg = d0_ref[:, 0:1] + d1_ref[:, 0:1]
    nrm = lax.rsqrt(jnp.maximum(deg, 1.0))
    n_ref[...] = nrm
    hs_ref[...] = h_ref[...] * nrm


def _norm(d0, d1, h):
    return pl.pallas_call(
        _norm_body,
        grid=(_GRID,),
        in_specs=[
            pl.BlockSpec((_BLK, DEGW), lambda i: (i, 0)),
            pl.BlockSpec((_BLK, DEGW), lambda i: (i, 0)),
            pl.BlockSpec((_BLK, HID), lambda i: (i, 0)),
        ],
        out_specs=[
            pl.BlockSpec((_BLK, 1), lambda i: (i, 0)),
            pl.BlockSpec((_BLK, HID), lambda i: (i, 0)),
        ],
        out_shape=[
            jax.ShapeDtypeStruct((N, 1), jnp.float32),
            jax.ShapeDtypeStruct((N, HID), jnp.float32),
        ],
    )(d0, d1, h)


def _combine_body(beta, a0_ref, a1_ref, h0_ref, n_ref, w_ref, h_ref, hs_ref):
    agg = (a0_ref[...] + a1_ref[...]) * n_ref[...]
    r = (1.0 - ALPHA) * agg + ALPHA * h0_ref[...]
    r = (1.0 - beta) * r + beta * jnp.dot(
        r, w_ref[...], preferred_element_type=jnp.float32)
    h = jnp.maximum(r, 0.0)
    h_ref[...] = h
    hs_ref[...] = h * n_ref[...]


def _combine(a0, a1, h0, nrm, w, beta):
    return pl.pallas_call(
        functools.partial(_combine_body, beta),
        grid=(_GRID,),
        in_specs=[
            pl.BlockSpec((_BLK, HID), lambda i: (i, 0)),
            pl.BlockSpec((_BLK, HID), lambda i: (i, 0)),
            pl.BlockSpec((_BLK, HID), lambda i: (i, 0)),
            pl.BlockSpec((_BLK, 1), lambda i: (i, 0)),
            pl.BlockSpec((HID, HID), lambda i: (0, 0)),
        ],
        out_specs=[
            pl.BlockSpec((_BLK, HID), lambda i: (i, 0)),
            pl.BlockSpec((_BLK, HID), lambda i: (i, 0)),
        ],
        out_shape=[
            jax.ShapeDtypeStruct((N, HID), jnp.float32),
            jax.ShapeDtypeStruct((N, HID), jnp.float32),
        ],
    )(a0, a1, h0, nrm, w)


def _out_body(h_ref, w_ref, b_ref, o_ref):
    o = jnp.dot(h_ref[...], w_ref[...],
                preferred_element_type=jnp.float32) + b_ref[...]
    m = jnp.max(o, axis=1, keepdims=True)
    e = jnp.exp(o - m)
    lse = jnp.log(jnp.sum(e, axis=1, keepdims=True)) + m
    o_ref[...] = o - lse


def _outproj(h, wt, b):
    return pl.pallas_call(
        _out_body,
        grid=(_GRID,),
        in_specs=[
            pl.BlockSpec((_BLK, HID), lambda i: (i, 0)),
            pl.BlockSpec((HID, OUT), lambda i: (0, 0)),
            pl.BlockSpec((1, OUT), lambda i: (0, 0)),
        ],
        out_specs=pl.BlockSpec((_BLK, OUT), lambda i: (i, 0)),
        out_shape=jax.ShapeDtypeStruct((N, OUT), jnp.float32),
    )(h, wt, b)


def kernel(features, edge_index, fc0_w, fc0_b, fc1_w, fc1_b, conv_w):
    src = edge_index[0]
    dst = edge_index[1]

    pad = EPAD - E
    src_p = jnp.concatenate([src, jnp.zeros((pad,), jnp.int32)])
    dst_p = jnp.concatenate([dst, jnp.full((pad,), N, jnp.int32)])
    packed = ((src_p << PACK_SHIFT) | dst_p).reshape(NW, NCHUNK, CH)
    zeros = jnp.zeros((SLAB, HID), jnp.float32)

    deg2 = _deg_call(packed, zeros)
    h0 = _proj(features, fc0_w.T, fc0_b.reshape(1, HID))
    nrm, hs = _norm(deg2[0], deg2[1], h0)

    h = h0
    for i in range(L_LAYERS):
        agg2 = _spmm_call(hs, src, dst, zeros)
        beta = math.log(LAM / (i + 1) + 1.0)
        h, hs = _combine(agg2[0], agg2[1], h0, nrm, conv_w[i], beta)

    return _outproj(h, fc1_w.T, fc1_b.reshape(1, OUT))


# speedup vs baseline: 3.2651x; 1.4271x over previous
import functools
import math

import jax
import jax.numpy as jnp
from jax import lax
from jax.experimental import pallas as pl
from jax.experimental.pallas import tpu as pltpu
from jax.experimental.pallas import tpu_sc as plsc

N = 10000
E = 320000
IN = 128
HID = 128
OUT = 64
L_LAYERS = 8
ALPHA = 0.1
LAM = 0.5

NC = 2
NS = 16
NW = NC * NS
CH = 128
NCHUNK = 80
EPT = NCHUNK * CH
EPAD = NW * EPT
NPAD = N + 8
NSLAB = 10
SLAB = N // NSLAB
ZR = 200
DEGW = 128

_mesh = plsc.VectorSubcoreMesh(
    core_axis_name="c", subcore_axis_name="s", num_cores=NC, num_subcores=NS)


PACK_SHIFT = 14
PACK_MASK = (1 << PACK_SHIFT) - 1


def _unpack_chunk(packed, j, sidx, didx):
    for k in range(CH // 16):
        p = packed[j, pl.ds(16 * k, 16)]
        if sidx is not None:
            sidx[pl.ds(16 * k, 16)] = p >> PACK_SHIFT
        didx[pl.ds(16 * k, 16)] = p & PACK_MASK


def _deg_body(packed_hbm, zeros_hbm, out_hbm, packed_v, didx2, ones_v, deg_s,
              ssem):
    cid = lax.axis_index("c")
    sid = lax.axis_index("s")
    wid = sid * NC + cid

    one16 = jnp.ones((16,), jnp.float32)

    def fill_ones(i, _):
        for k in range(DEGW // 16):
            ones_v[i, pl.ds(16 * k, 16)] = one16
        return 0

    lax.fori_loop(0, CH, fill_ones, 0, unroll=False)

    @pl.when(sid < NSLAB)
    def _():
        pltpu.sync_copy(zeros_hbm, deg_s.at[pl.ds(sid * SLAB, SLAB)])

    pltpu.sync_copy(packed_hbm.at[wid], packed_v)

    def unpack(j, _):
        _unpack_chunk(packed_v, j, None, didx2.at[j])
        return 0

    lax.fori_loop(0, NCHUNK, unpack, 0, unroll=False)
    plsc.subcore_barrier()

    def chunk(j, _):
        pltpu.async_copy(ones_v, deg_s.at[didx2.at[j]], ssem, add=True)
        return 0

    lax.fori_loop(0, NCHUNK, chunk, 0, unroll=False)

    def drain(j, _):
        pltpu.make_async_copy(ones_v, deg_s.at[didx2.at[j]], ssem).wait()
        return 0

    lax.fori_loop(0, NCHUNK, drain, 0, unroll=False)
    plsc.subcore_barrier()

    @pl.when(sid < NSLAB)
    def _():
        pltpu.sync_copy(deg_s.at[pl.ds(sid * SLAB, SLAB)],
                        out_hbm.at[cid, pl.ds(sid * SLAB, SLAB)])


_deg_call = pl.kernel(
    _deg_body,
    out_type=jax.ShapeDtypeStruct((NC, N, DEGW), jnp.float32),
    mesh=_mesh,
    scratch_types=[
        pltpu.VMEM((NCHUNK, CH), jnp.int32),
        pltpu.VMEM((NCHUNK, CH), jnp.int32),
        pltpu.VMEM((CH, DEGW), jnp.float32),
        pltpu.VMEM_SHARED((NPAD, DEGW), jnp.float32),
        pltpu.SemaphoreType.DMA,
    ],
)


SCH = 80
SNCHUNK = 10000 // SCH
SEPT = 10000


def _spmm_body(hs_hbm, src_hbm, dst_hbm, zeros_hbm, out_hbm,
               sidx0, sidx1, sidx2, didx0, didx1, didx2,
               rows0, rows1, rows2,
               agg_s, is0, is1, is2, gs0, gs1, gs2):
    cid = lax.axis_index("c")
    sid = lax.axis_index("s")
    wid = sid * NC + cid

    @pl.when(sid < NSLAB)
    def _():
        pltpu.sync_copy(zeros_hbm, agg_s.at[pl.ds(sid * SLAB, SLAB)])
    plsc.subcore_barrier()

    rows = (rows0, rows1, rows2)
    sidx = (sidx0, sidx1, sidx2)
    didx = (didx0, didx1, didx2)
    isem = (is0, is1, is2)
    gsem = (gs0, gs1, gs2)
    base = wid * SEPT

    def idx_fire(j, b):
        pltpu.async_copy(src_hbm.at[pl.ds(base + j * SCH, SCH)],
                         sidx[b], isem[b])
        pltpu.async_copy(dst_hbm.at[pl.ds(base + j * SCH, SCH)],
                         didx[b], isem[b])

    def idx_wait(j, b):
        pltpu.make_async_copy(src_hbm.at[pl.ds(base + j * SCH, SCH)],
                              sidx[b], isem[b]).wait()
        pltpu.make_async_copy(dst_hbm.at[pl.ds(base + j * SCH, SCH)],
                              didx[b], isem[b]).wait()

    def gather_fire(b):
        pltpu.async_copy(hs_hbm.at[sidx[b]], rows[b], gsem[b])

    def finish(b):
        pltpu.make_async_copy(hs_hbm.at[sidx[b]], rows[b], gsem[b]).wait()
        pltpu.sync_copy(rows[b], agg_s.at[didx[b]], add=True)

    idx_fire(0, 0)
    idx_wait(0, 0)
    gather_fire(0)
    idx_fire(1, 1)

    def chunk3(j3, _):
        for m in range(3):
            j = 1 + j3 * 3 + m
            b = (1 + m) % 3
            idx_wait(j, b)
            gather_fire(b)
            idx_fire(j + 1, (b + 1) % 3)
            finish((b + 2) % 3)
        return 0

    lax.fori_loop(0, (SNCHUNK - 2) // 3, chunk3, 0, unroll=False)
    jl = SNCHUNK - 1
    bl = jl % 3
    idx_wait(jl, bl)
    gather_fire(bl)
    finish((bl + 2) % 3)
    finish(bl)
    plsc.subcore_barrier()

    @pl.when(sid < NSLAB)
    def _():
        pltpu.sync_copy(agg_s.at[pl.ds(sid * SLAB, SLAB)],
                        out_hbm.at[cid, pl.ds(sid * SLAB, SLAB)])


_spmm_call = pl.kernel(
    _spmm_body,
    out_type=jax.ShapeDtypeStruct((NC, N, HID), jnp.float32),
    mesh=_mesh,
    scratch_types=[
        pltpu.VMEM((SCH,), jnp.int32),
        pltpu.VMEM((SCH,), jnp.int32),
        pltpu.VMEM((SCH,), jnp.int32),
        pltpu.VMEM((SCH,), jnp.int32),
        pltpu.VMEM((SCH,), jnp.int32),
        pltpu.VMEM((SCH,), jnp.int32),
        pltpu.VMEM((SCH, HID), jnp.float32),
        pltpu.VMEM((SCH, HID), jnp.float32),
        pltpu.VMEM((SCH, HID), jnp.float32),
        pltpu.VMEM_SHARED((N, HID), jnp.float32),
        pltpu.SemaphoreType.DMA,
        pltpu.SemaphoreType.DMA,
        pltpu.SemaphoreType.DMA,
        pltpu.SemaphoreType.DMA,
        pltpu.SemaphoreType.DMA,
        pltpu.SemaphoreType.DMA,
    ],
)


_BLK = 1000
_GRID = N // _BLK


def _proj_body(x_ref, w_ref, b_ref, o_ref):
    o_ref[...] = jnp.maximum(
        jnp.dot(x_ref[...], w_ref[...], preferred_element_type=jnp.float32)
        + b_ref[...], 0.0)


def _proj(x, wt, b):
    return pl.pallas_call(
        _proj_body,
        grid=(_GRID,),
        in_specs=[
            pl.BlockSpec((_BLK, IN), lambda i: (i, 0)),
            pl.BlockSpec((IN, HID), lambda i: (0, 0)),
            pl.BlockSpec((1, HID), lambda i: (0, 0)),
        ],
        out_specs=pl.BlockSpec((_BLK, HID), lambda i: (i, 0)),
        out_shape=jax.ShapeDtypeStruct((N, HID), jnp.float32),
    )(x, wt, b)


def _norm_body(d0_ref, d1_ref, h_ref, n_ref, hs_ref):
    deg = d0_ref[:, 0:1] + d1_ref[:, 0:1]
    nrm = lax.rsqrt(jnp.maximum(deg, 1.0))
    n_ref[...] = nrm
    hs_ref[...] = h_ref[...] * nrm


def _norm(d0, d1, h):
    return pl.pallas_call(
        _norm_body,
        grid=(_GRID,),
        in_specs=[
            pl.BlockSpec((_BLK, DEGW), lambda i: (i, 0)),
            pl.BlockSpec((_BLK, DEGW), lambda i: (i, 0)),
            pl.BlockSpec((_BLK, HID), lambda i: (i, 0)),
        ],
        out_specs=[
            pl.BlockSpec((_BLK, 1), lambda i: (i, 0)),
            pl.BlockSpec((_BLK, HID), lambda i: (i, 0)),
        ],
        out_shape=[
            jax.ShapeDtypeStruct((N, 1), jnp.float32),
            jax.ShapeDtypeStruct((N, HID), jnp.float32),
        ],
    )(d0, d1, h)


def _combine_body(beta, a0_ref, a1_ref, h0_ref, n_ref, w_ref, h_ref, hs_ref):
    agg = (a0_ref[...] + a1_ref[...]) * n_ref[...]
    r = (1.0 - ALPHA) * agg + ALPHA * h0_ref[...]
    r = (1.0 - beta) * r + beta * jnp.dot(
        r, w_ref[...], preferred_element_type=jnp.float32)
    h = jnp.maximum(r, 0.0)
    h_ref[...] = h
    hs_ref[...] = h * n_ref[...]


def _combine(a0, a1, h0, nrm, w, beta):
    return pl.pallas_call(
        functools.partial(_combine_body, beta),
        grid=(_GRID,),
        in_specs=[
            pl.BlockSpec((_BLK, HID), lambda i: (i, 0)),
            pl.BlockSpec((_BLK, HID), lambda i: (i, 0)),
            pl.BlockSpec((_BLK, HID), lambda i: (i, 0)),
            pl.BlockSpec((_BLK, 1), lambda i: (i, 0)),
            pl.BlockSpec((HID, HID), lambda i: (0, 0)),
        ],
        out_specs=[
            pl.BlockSpec((_BLK, HID), lambda i: (i, 0)),
            pl.BlockSpec((_BLK, HID), lambda i: (i, 0)),
        ],
        out_shape=[
            jax.ShapeDtypeStruct((N, HID), jnp.float32),
            jax.ShapeDtypeStruct((N, HID), jnp.float32),
        ],
    )(a0, a1, h0, nrm, w)


def _out_body(h_ref, w_ref, b_ref, o_ref):
    o = jnp.dot(h_ref[...], w_ref[...],
                preferred_element_type=jnp.float32) + b_ref[...]
    m = jnp.max(o, axis=1, keepdims=True)
    e = jnp.exp(o - m)
    lse = jnp.log(jnp.sum(e, axis=1, keepdims=True)) + m
    o_ref[...] = o - lse


def _outproj(h, wt, b):
    return pl.pallas_call(
        _out_body,
        grid=(_GRID,),
        in_specs=[
            pl.BlockSpec((_BLK, HID), lambda i: (i, 0)),
            pl.BlockSpec((HID, OUT), lambda i: (0, 0)),
            pl.BlockSpec((1, OUT), lambda i: (0, 0)),
        ],
        out_specs=pl.BlockSpec((_BLK, OUT), lambda i: (i, 0)),
        out_shape=jax.ShapeDtypeStruct((N, OUT), jnp.float32),
    )(h, wt, b)


def kernel(features, edge_index, fc0_w, fc0_b, fc1_w, fc1_b, conv_w):
    src = edge_index[0]
    dst = edge_index[1]

    pad = EPAD - E
    src_p = jnp.concatenate([src, jnp.zeros((pad,), jnp.int32)])
    dst_p = jnp.concatenate([dst, jnp.full((pad,), N, jnp.int32)])
    packed = ((src_p << PACK_SHIFT) | dst_p).reshape(NW, NCHUNK, CH)
    zeros = jnp.zeros((SLAB, HID), jnp.float32)

    deg2 = _deg_call(packed, zeros)
    h0 = _proj(features, fc0_w.T, fc0_b.reshape(1, HID))
    nrm, hs = _norm(deg2[0], deg2[1], h0)

    h = h0
    for i in range(L_LAYERS):
        agg2 = _spmm_call(hs, src, dst, zeros)
        beta = math.log(LAM / (i + 1) + 1.0)
        h, hs = _combine(agg2[0], agg2[1], h0, nrm, conv_w[i], beta)

    return _outproj(h, fc1_w.T, fc1_b.reshape(1, OUT))
